# baseline (speedup 1.0000x reference)
import functools

import jax
import jax.numpy as jnp
from jax import lax
from jax.experimental import pallas as pl
from jax.experimental.pallas import tpu as pltpu
from jax.experimental.pallas import tpu_sc as plsc

BATCH = 4096
SEQ = 200
EMB_DIM = 64

NC = 2
NS = 16
NW = NC * NS

BB = BATCH // NW
NBUF = 4
PAD = NBUF
NSTEPS = SEQ
NTILES = EMB_DIM // 8
TB_PAD = BB + 1


def _make_gather():
    mesh = plsc.VectorSubcoreMesh(core_axis_name="c", subcore_axis_name="s")

    @functools.partial(
        pl.kernel,
        mesh=mesh,
        out_type=jax.ShapeDtypeStruct((SEQ, NTILES, NW, 8, BB), jnp.float32),
        scratch_types=[
            pltpu.VMEM((NSTEPS + PAD, BB), jnp.int32),
            pltpu.VMEM((BB, EMB_DIM), jnp.float32),
            pltpu.VMEM((BB, EMB_DIM), jnp.float32),
            pltpu.VMEM((BB, EMB_DIM), jnp.float32),
            pltpu.VMEM((BB, EMB_DIM), jnp.float32),
            pltpu.VMEM((EMB_DIM, TB_PAD), jnp.float32),
            pltpu.VMEM((EMB_DIM, TB_PAD), jnp.float32),
            pltpu.VMEM((EMB_DIM, TB_PAD), jnp.float32),
            pltpu.VMEM((EMB_DIM, TB_PAD), jnp.float32),
            pltpu.SemaphoreType.DMA((NBUF,)),
            pltpu.SemaphoreType.DMA((NBUF,)),
        ],
        compiler_params=pltpu.CompilerParams(
            use_tc_tiling_on_sc=False, needs_layout_passes=False),
    )
    def gather_kernel(table_hbm, ids_hbm, out_hbm,
                      idx_v, g0, g1, g2, g3, t0, t1, t2, t3, gsem, osem):
        gbufs = [g0, g1, g2, g3]
        tbufs = [t0, t1, t2, t3]
        wid = lax.axis_index("s") * NC + lax.axis_index("c")
        pltpu.sync_copy(ids_hbm.at[wid], idx_v)
        iota = lax.iota(jnp.int32, 16)
        e_idx = [iota + 16 * k for k in range(EMB_DIM // 16)]

        def start_gather(g, sl):
            pltpu.async_copy(table_hbm.at[idx_v.at[g]], gbufs[sl], gsem.at[sl])

        def wait_gather(sl):
            pltpu.make_async_copy(
                table_hbm.at[pl.ds(0, BB)], gbufs[sl], gsem.at[sl]).wait()

        def transpose(sl):
            src, dst = gbufs[sl], tbufs[sl]

            @plsc.parallel_loop(0, BB, step=1, unroll=16)
            def body(b):
                b_idx = jnp.full((16,), b, jnp.int32)
                for k in range(EMB_DIM // 16):
                    v = src[b, pl.ds(16 * k, 16)]
                    plsc.store_scatter(dst, [e_idx[k], b_idx], v)

        def start_outs(g, sl):
            for te in range(NTILES):
                pltpu.async_copy(
                    tbufs[sl].at[pl.ds(te * 8, 8), pl.ds(0, BB)],
                    out_hbm.at[g, te, wid], osem.at[sl])

        def wait_outs(sl):
            for te in range(NTILES):
                pltpu.make_async_copy(
                    tbufs[sl].at[pl.ds(te * 8, 8), pl.ds(0, BB)],
                    out_hbm.at[0, 0, 0], osem.at[sl]).wait()

        for g in range(NBUF):
            start_gather(g, g % NBUF)
        for g in range(NBUF):
            sl = g % NBUF
            wait_gather(sl)
            transpose(sl)
            start_outs(g, sl)
            start_gather(g + NBUF, sl)

        def blk_body(blk, carry):
            for b in range(NBUF):
                g = NBUF + blk * NBUF + b
                sl = b
                wait_gather(sl)
                wait_outs(sl)
                transpose(sl)
                start_outs(g, sl)
                start_gather(g + NBUF, sl)
            return carry

        lax.fori_loop(0, (NSTEPS - NBUF) // NBUF, blk_body, 0)

        for sl in range(NBUF):
            wait_gather(sl)
            wait_outs(sl)

    return gather_kernel


_gather = _make_gather()


def kernel(input_ids, weight):
    ids = input_ids.astype(jnp.int32)
    ids_sc = ids.T.reshape(SEQ, NW, BB).transpose(1, 0, 2)
    ids_sc = jnp.concatenate(
        [ids_sc, jnp.zeros((NW, PAD, BB), jnp.int32)], axis=1)
    out = _gather(weight, ids_sc)
    return out.transpose(2, 4, 0, 1, 3).reshape(BATCH, SEQ, EMB_DIM)


# speedup vs baseline: 1.0334x; 1.0334x over previous
import functools

import jax
import jax.numpy as jnp
from jax import lax
from jax.experimental import pallas as pl
from jax.experimental.pallas import tpu as pltpu
from jax.experimental.pallas import tpu_sc as plsc

BATCH = 4096
SEQ = 200
EMB_DIM = 64

NC = 2
NS = 16
NW = NC * NS

BB = BATCH // NW
NBUF = 4
PAD = NBUF
NSTEPS = SEQ
NTILES = EMB_DIM // 8
TB_PAD = BB + 1


def _make_gather():
    mesh = plsc.VectorSubcoreMesh(core_axis_name="c", subcore_axis_name="s")

    @functools.partial(
        pl.kernel,
        mesh=mesh,
        out_type=jax.ShapeDtypeStruct((SEQ, NTILES, NW, 8, BB), jnp.float32),
        scratch_types=[
            pltpu.VMEM((SEQ // 8 + 1, 8, BB), jnp.int32),
            pltpu.VMEM((BB, EMB_DIM), jnp.float32),
            pltpu.VMEM((BB, EMB_DIM), jnp.float32),
            pltpu.VMEM((BB, EMB_DIM), jnp.float32),
            pltpu.VMEM((BB, EMB_DIM), jnp.float32),
            pltpu.VMEM((EMB_DIM, TB_PAD), jnp.float32),
            pltpu.VMEM((EMB_DIM, TB_PAD), jnp.float32),
            pltpu.VMEM((EMB_DIM, TB_PAD), jnp.float32),
            pltpu.VMEM((EMB_DIM, TB_PAD), jnp.float32),
            pltpu.SemaphoreType.DMA((NBUF,)),
            pltpu.SemaphoreType.DMA((NBUF,)),
        ],
        compiler_params=pltpu.CompilerParams(
            use_tc_tiling_on_sc=False, needs_layout_passes=False),
    )
    def gather_kernel(table_hbm, ids_hbm, out_hbm,
                      idx_v, g0, g1, g2, g3, t0, t1, t2, t3, gsem, osem):
        gbufs = [g0, g1, g2, g3]
        tbufs = [t0, t1, t2, t3]
        wid = lax.axis_index("s") * NC + lax.axis_index("c")
        pltpu.sync_copy(ids_hbm.at[:, wid], idx_v.at[pl.ds(0, SEQ // 8)])
        zeros16 = jnp.zeros((16,), jnp.int32)
        for s8 in range(8):
            for j in range(BB // 16):
                idx_v[SEQ // 8, s8, pl.ds(16 * j, 16)] = zeros16
        iota = lax.iota(jnp.int32, 16)
        e_idx = [iota + 16 * k for k in range(EMB_DIM // 16)]

        def start_gather(g, sl):
            pltpu.async_copy(table_hbm.at[idx_v.at[g // 8, g % 8]],
                             gbufs[sl], gsem.at[sl])

        def wait_gather(sl):
            pltpu.make_async_copy(
                table_hbm.at[pl.ds(0, BB)], gbufs[sl], gsem.at[sl]).wait()

        def transpose(sl):
            src, dst = gbufs[sl], tbufs[sl]

            @plsc.parallel_loop(0, BB, step=1, unroll=8)
            def body(b):
                b_idx = jnp.full((16,), b, jnp.int32)
                for k in range(EMB_DIM // 16):
                    v = src[b, pl.ds(16 * k, 16)]
                    plsc.store_scatter(dst, [e_idx[k], b_idx], v)

        def start_outs(g, sl):
            for te in range(NTILES):
                pltpu.async_copy(
                    tbufs[sl].at[pl.ds(te * 8, 8), pl.ds(0, BB)],
                    out_hbm.at[g, te, wid], osem.at[sl])

        def wait_outs(sl):
            for te in range(NTILES):
                pltpu.make_async_copy(
                    tbufs[sl].at[pl.ds(te * 8, 8), pl.ds(0, BB)],
                    out_hbm.at[0, 0, 0], osem.at[sl]).wait()

        for g in range(NBUF):
            start_gather(g, g % NBUF)
        for g in range(NBUF):
            sl = g % NBUF
            wait_gather(sl)
            transpose(sl)
            start_outs(g, sl)
            start_gather(g + NBUF, sl)

        def blk_body(blk, carry):
            for b in range(NBUF):
                g = NBUF + blk * NBUF + b
                sl = b
                wait_gather(sl)
                wait_outs(sl)
                transpose(sl)
                start_outs(g, sl)
                start_gather(g + NBUF, sl)
            return carry

        lax.fori_loop(0, (NSTEPS - NBUF) // NBUF, blk_body, 0)

        for sl in range(NBUF):
            wait_gather(sl)
            wait_outs(sl)

    return gather_kernel


_gather = _make_gather()


def kernel(input_ids, weight):
    ids_t = (input_ids.astype(jnp.int32).T
             .reshape(SEQ // 8, 8, NW, BB).transpose(0, 2, 1, 3))
    out = _gather(weight, ids_t)
    return out.transpose(2, 4, 0, 1, 3).reshape(BATCH, SEQ, EMB_DIM)


# speedup vs baseline: 1.0339x; 1.0004x over previous
import functools

import jax
import jax.numpy as jnp
from jax import lax
from jax.experimental import pallas as pl
from jax.experimental.pallas import tpu as pltpu
from jax.experimental.pallas import tpu_sc as plsc

BATCH = 4096
SEQ = 200
EMB_DIM = 64

NC = 2
NS = 16
NW = NC * NS

BB = BATCH // NW
NBUF = 4
PAD = NBUF
NSTEPS = SEQ
NTILES = EMB_DIM // 8
TB_PAD = BB + 1


def _make_gather():
    mesh = plsc.VectorSubcoreMesh(core_axis_name="c", subcore_axis_name="s")

    @functools.partial(
        pl.kernel,
        mesh=mesh,
        out_type=jax.ShapeDtypeStruct((SEQ, NTILES, NW, 8, BB), jnp.float32),
        scratch_types=[
            pltpu.VMEM((SEQ // 8 + 1, 8, BB), jnp.int32),
            pltpu.VMEM((BB, EMB_DIM), jnp.float32),
            pltpu.VMEM((BB, EMB_DIM), jnp.float32),
            pltpu.VMEM((BB, EMB_DIM), jnp.float32),
            pltpu.VMEM((BB, EMB_DIM), jnp.float32),
            pltpu.VMEM((EMB_DIM, TB_PAD), jnp.float32),
            pltpu.VMEM((EMB_DIM, TB_PAD), jnp.float32),
            pltpu.VMEM((EMB_DIM, TB_PAD), jnp.float32),
            pltpu.VMEM((EMB_DIM, TB_PAD), jnp.float32),
            pltpu.SemaphoreType.DMA((NBUF,)),
            pltpu.SemaphoreType.DMA((NBUF,)),
        ],
        compiler_params=pltpu.CompilerParams(
            use_tc_tiling_on_sc=False, needs_layout_passes=False),
    )
    def gather_kernel(table_hbm, ids_hbm, out_hbm,
                      idx_v, g0, g1, g2, g3, t0, t1, t2, t3, gsem, osem):
        gbufs = [g0, g1, g2, g3]
        tbufs = [t0, t1, t2, t3]
        wid = lax.axis_index("s") * NC + lax.axis_index("c")
        pltpu.sync_copy(ids_hbm.at[:, wid], idx_v.at[pl.ds(0, SEQ // 8)])
        zeros16 = jnp.zeros((16,), jnp.int32)
        for s8 in range(8):
            for j in range(BB // 16):
                idx_v[SEQ // 8, s8, pl.ds(16 * j, 16)] = zeros16
        iota = lax.iota(jnp.int32, 16)
        e_idx = [iota + 16 * k for k in range(EMB_DIM // 16)]

        def start_gather(g, sl):
            pltpu.async_copy(table_hbm.at[idx_v.at[g // 8, g % 8]],
                             gbufs[sl], gsem.at[sl])

        def wait_gather(sl):
            pltpu.make_async_copy(
                table_hbm.at[pl.ds(0, BB)], gbufs[sl], gsem.at[sl]).wait()

        def transpose(sl):
            src, dst = gbufs[sl], tbufs[sl]

            @plsc.parallel_loop(0, BB, step=1, unroll=4)
            def body(b):
                b_idx = jnp.full((16,), b, jnp.int32)
                for k in range(EMB_DIM // 16):
                    v = src[b, pl.ds(16 * k, 16)]
                    plsc.store_scatter(dst, [e_idx[k], b_idx], v)

        def start_outs(g, sl):
            for te in range(NTILES):
                pltpu.async_copy(
                    tbufs[sl].at[pl.ds(te * 8, 8), pl.ds(0, BB)],
                    out_hbm.at[g, te, wid], osem.at[sl])

        def wait_outs(sl):
            for te in range(NTILES):
                pltpu.make_async_copy(
                    tbufs[sl].at[pl.ds(te * 8, 8), pl.ds(0, BB)],
                    out_hbm.at[0, 0, 0], osem.at[sl]).wait()

        for g in range(NBUF):
            start_gather(g, g % NBUF)
        for g in range(NBUF):
            sl = g % NBUF
            wait_gather(sl)
            transpose(sl)
            start_outs(g, sl)
            start_gather(g + NBUF, sl)

        def blk_body(blk, carry):
            for b in range(NBUF):
                g = NBUF + blk * NBUF + b
                sl = b
                wait_gather(sl)
                wait_outs(sl)
                transpose(sl)
                start_outs(g, sl)
                start_gather(g + NBUF, sl)
            return carry

        lax.fori_loop(0, (NSTEPS - NBUF) // NBUF, blk_body, 0)

        for sl in range(NBUF):
            wait_gather(sl)
            wait_outs(sl)

    return gather_kernel


_gather = _make_gather()


def kernel(input_ids, weight):
    ids_t = (input_ids.astype(jnp.int32).T
             .reshape(SEQ // 8, 8, NW, BB).transpose(0, 2, 1, 3))
    out = _gather(weight, ids_t)
    return out.transpose(2, 4, 0, 1, 3).reshape(BATCH, SEQ, EMB_DIM)
